# baseline (speedup 1.0000x reference)
import jax
import jax.numpy as jnp
from jax import lax
from jax.experimental import pallas as pl
from jax.experimental.pallas import tpu as pltpu
from jax.experimental.pallas import tpu_sc as plsc

N = 50000
E = 800000
FIN = 784
H = 64
C = 20
ALPHA = 0.2
BETA = 0.05
C1 = (1.0 - BETA) * (1.0 - ALPHA)
C2 = (1.0 - ALPHA) * BETA
C3 = (1.0 - BETA) * ALPHA
C4 = BETA * ALPHA

NPAD = 51200
TROWS = NPAD // 16
LAST0 = 15 * TROWS
LASTN = N - LAST0
DUMP_D = N
CH = 256
NCHUNK = E // CH
BM = 1000

_MESH = plsc.VectorSubcoreMesh(core_axis_name="c", subcore_axis_name="s")


def _splat(v, j):
    idx = jnp.full((16, 1), j, dtype=jnp.int32)
    dn = lax.GatherDimensionNumbers(
        offset_dims=(), collapsed_slice_dims=(0,), start_index_map=(0,))
    return lax.gather(v, idx, dn, (1,),
                      mode=lax.GatherScatterMode.PROMISE_IN_BOUNDS)


def _deg_body(row_hbm, col_hbm, attr_hbm, z1_hbm, n1_hbm,
              dega_out, degb_out, lwa_out, lwb_out,
              deg_t, lw_t, rowb0, rowb1, colb, attrb, vdeg, ilw0, ilw1, sem):
    c = lax.axis_index("c")
    s = lax.axis_index("s")
    w = s * 2 + c
    r0 = s * TROWS
    pltpu.sync_copy(z1_hbm, deg_t.at[pl.ds(r0, TROWS)])
    pltpu.sync_copy(n1_hbm, lw_t.at[pl.ds(r0, TROWS)])
    plsc.subcore_barrier()

    dumpv = jnp.full((16,), DUMP_D, jnp.int32)
    nt = (NCHUNK - w + 31) // 32

    def chunk(k, _):
        off = (w + k * 32) * CH
        cp1 = pltpu.async_copy(row_hbm.at[pl.ds(off, 128)], rowb0, sem)
        cp2 = pltpu.async_copy(row_hbm.at[pl.ds(off + 128, 128)], rowb1, sem)
        cp3 = pltpu.async_copy(col_hbm.at[pl.ds(off, CH)], colb, sem)
        cp4 = pltpu.async_copy(attr_hbm.at[pl.ds(off, CH)], attrb, sem)
        cp1.wait()
        cp2.wait()
        cp3.wait()
        cp4.wait()

        for g in range(16):
            h = (g % 8) * 16
            rsrc = rowb0 if g < 8 else rowb1
            rv = rsrc[pl.ds(h, 16)]
            cv = colb[pl.ds(g * 16, 16)]
            wv = attrb[pl.ds(g * 16, 16)]
            selfm = rv == cv
            ew = jnp.where(selfm, 0.0, wv)
            lwi = jnp.where(selfm, rv, dumpv)
            vdeg[pl.ds(g * 16, 16)] = ew
            if g < 8:
                ilw0[pl.ds(h, 16)] = lwi
            else:
                ilw1[pl.ds(h, 16)] = lwi

        d1 = pltpu.async_copy(vdeg.at[pl.ds(0, 128)],
                              deg_t.at[rowb0], sem, add=True)
        d2 = pltpu.async_copy(vdeg.at[pl.ds(128, 128)],
                              deg_t.at[rowb1], sem, add=True)
        d3 = pltpu.async_copy(attrb.at[pl.ds(0, 128)], lw_t.at[ilw0], sem)
        d4 = pltpu.async_copy(attrb.at[pl.ds(128, 128)], lw_t.at[ilw1], sem)
        d1.wait()
        d2.wait()
        d3.wait()
        d4.wait()
        return 0
    lax.fori_loop(0, nt, chunk, 0)
    plsc.subcore_barrier()

    @pl.when(c == 0)
    def _():
        pltpu.sync_copy(deg_t.at[pl.ds(r0, TROWS)],
                        dega_out.at[pl.ds(r0, TROWS)])
        pltpu.sync_copy(lw_t.at[pl.ds(r0, TROWS)],
                        lwa_out.at[pl.ds(r0, TROWS)])

    @pl.when(c == 1)
    def _():
        pltpu.sync_copy(deg_t.at[pl.ds(r0, TROWS)],
                        degb_out.at[pl.ds(r0, TROWS)])
        pltpu.sync_copy(lw_t.at[pl.ds(r0, TROWS)],
                        lwb_out.at[pl.ds(r0, TROWS)])


_deg_pass = pl.kernel(
    _deg_body,
    out_type=[jax.ShapeDtypeStruct((NPAD,), jnp.float32),
              jax.ShapeDtypeStruct((NPAD,), jnp.float32),
              jax.ShapeDtypeStruct((NPAD,), jnp.float32),
              jax.ShapeDtypeStruct((NPAD,), jnp.float32)],
    mesh=_MESH,
    scratch_types=[
        pltpu.VMEM_SHARED((NPAD,), jnp.float32),
        pltpu.VMEM_SHARED((NPAD,), jnp.float32),
        pltpu.VMEM((128,), jnp.int32),
        pltpu.VMEM((128,), jnp.int32),
        pltpu.VMEM((CH,), jnp.int32),
        pltpu.VMEM((CH,), jnp.float32),
        pltpu.VMEM((CH,), jnp.float32),
        pltpu.VMEM((128,), jnp.int32),
        pltpu.VMEM((128,), jnp.int32),
        pltpu.SemaphoreType.DMA,
    ],
)


def _scat_body(row_hbm, col_hbm, attr_hbm, sq0, sq1, sq2, sq3, agg_out,
               acc0, acc1,
               rA0, rA1, cA0, cA1, aA, gbA,
               rB0, rB1, cB0, cB1, aB, gbB,
               semEA, semEB, semGA, semGB, semSA, semSB):
    c = lax.axis_index("c")
    s = lax.axis_index("s")
    r0 = s * TROWS

    A = (rA0, rA1, cA0, cA1, aA, gbA, semEA, semGA, semSA)
    B = (rB0, rB1, cB0, cB1, aB, gbB, semEB, semGB, semSB)

    def fire_edges(S, k):
        off = (s + k * 16) * CH
        pltpu.async_copy(row_hbm.at[pl.ds(off, 128)], S[0], S[6])
        pltpu.async_copy(row_hbm.at[pl.ds(off + 128, 128)], S[1], S[6])
        pltpu.async_copy(col_hbm.at[pl.ds(off, 128)], S[2], S[6])
        pltpu.async_copy(col_hbm.at[pl.ds(off + 128, 128)], S[3], S[6])
        pltpu.async_copy(attr_hbm.at[pl.ds(off, CH)], S[4], S[6])

    def wait_edges(S):
        pltpu.make_async_copy(row_hbm.at[pl.ds(0, 128)], S[0], S[6]).wait()
        pltpu.make_async_copy(row_hbm.at[pl.ds(0, 128)], S[1], S[6]).wait()
        pltpu.make_async_copy(col_hbm.at[pl.ds(0, 128)], S[2], S[6]).wait()
        pltpu.make_async_copy(col_hbm.at[pl.ds(0, 128)], S[3], S[6]).wait()
        pltpu.make_async_copy(attr_hbm.at[pl.ds(0, CH)], S[4], S[6]).wait()

    def fire_gathers(S):
        gb = S[5]

        @pl.when(c == 0)
        def _():
            pltpu.async_copy(sq0.at[S[0]], gb.at[pl.ds(0, 128)], S[7])
            pltpu.async_copy(sq0.at[S[1]], gb.at[pl.ds(128, 128)], S[7])
            pltpu.async_copy(sq1.at[S[0]], gb.at[pl.ds(CH, 128)], S[7])
            pltpu.async_copy(sq1.at[S[1]], gb.at[pl.ds(CH + 128, 128)], S[7])

        @pl.when(c == 1)
        def _():
            pltpu.async_copy(sq2.at[S[0]], gb.at[pl.ds(0, 128)], S[7])
            pltpu.async_copy(sq2.at[S[1]], gb.at[pl.ds(128, 128)], S[7])
            pltpu.async_copy(sq3.at[S[0]], gb.at[pl.ds(CH, 128)], S[7])
            pltpu.async_copy(sq3.at[S[1]], gb.at[pl.ds(CH + 128, 128)], S[7])

    def wait_gathers(S):
        gb = S[5]
        pltpu.make_async_copy(sq0.at[S[0]], gb.at[pl.ds(0, 128)], S[7]).wait()
        pltpu.make_async_copy(sq0.at[S[1]], gb.at[pl.ds(128, 128)], S[7]).wait()
        pltpu.make_async_copy(sq1.at[S[0]], gb.at[pl.ds(CH, 128)], S[7]).wait()
        pltpu.make_async_copy(sq1.at[S[1]], gb.at[pl.ds(CH + 128, 128)], S[7]).wait()

    def scale(S):
        gb = S[5]
        for g in range(16):
            h = (g % 8) * 16
            rv = (S[0] if g < 8 else S[1])[pl.ds(h, 16)]
            cv = (S[2] if g < 8 else S[3])[pl.ds(h, 16)]
            wv = S[4][pl.ds(g * 16, 16)]
            ew = jnp.where(rv == cv, 0.0, wv)
            for j in range(16):
                e = g * 16 + j
                sc_ = _splat(ew, j)
                gb[e] = gb[e, pl.ds(0, 16)] * sc_
                gb[CH + e] = gb[CH + e, pl.ds(0, 16)] * sc_

    def fire_scatters(S):
        gb = S[5]
        pltpu.async_copy(gb.at[pl.ds(0, 128)], acc0.at[S[2]], S[8], add=True)
        pltpu.async_copy(gb.at[pl.ds(128, 128)], acc0.at[S[3]], S[8], add=True)
        pltpu.async_copy(gb.at[pl.ds(CH, 128)], acc1.at[S[2]], S[8], add=True)
        pltpu.async_copy(gb.at[pl.ds(CH + 128, 128)], acc1.at[S[3]], S[8], add=True)

    def wait_scatters(S):
        gb = S[5]
        pltpu.make_async_copy(gb.at[pl.ds(0, 128)], acc0.at[S[2]], S[8]).wait()
        pltpu.make_async_copy(gb.at[pl.ds(128, 128)], acc0.at[S[3]], S[8]).wait()
        pltpu.make_async_copy(gb.at[pl.ds(CH, 128)], acc1.at[S[2]], S[8]).wait()
        pltpu.make_async_copy(gb.at[pl.ds(CH + 128, 128)], acc1.at[S[3]], S[8]).wait()

    zv = jnp.zeros((16,), jnp.float32)
    for r in range(256):
        gbA[r] = zv
    for accx in (acc0, acc1):
        for kk in range(12):
            pltpu.sync_copy(gbA.at[pl.ds(0, 256)],
                            accx.at[pl.ds(r0 + kk * 256, 256)])
        pltpu.sync_copy(gbA.at[pl.ds(0, 128)],
                        accx.at[pl.ds(r0 + 3072, 128)])
    plsc.subcore_barrier()

    nt = (NCHUNK - s + 15) // 16
    npair = nt // 2
    rem = nt - 2 * npair

    fire_edges(A, 0)

    def pairbody(p, _):
        k0 = 2 * p
        wait_edges(A)
        fire_gathers(A)

        @pl.when(k0 >= 1)
        def _():
            wait_scatters(B)
        fire_edges(B, k0 + 1)
        wait_gathers(A)
        scale(A)
        fire_scatters(A)

        wait_edges(B)
        fire_gathers(B)
        wait_scatters(A)

        @pl.when(k0 + 2 < nt)
        def _():
            fire_edges(A, k0 + 2)
        wait_gathers(B)
        scale(B)
        fire_scatters(B)
        return 0
    lax.fori_loop(0, npair, pairbody, 0)

    @pl.when(rem == 1)
    def _():
        wait_edges(A)
        fire_gathers(A)
        wait_scatters(B)
        wait_gathers(A)
        scale(A)
        fire_scatters(A)
        wait_scatters(A)

    @pl.when(rem == 0)
    def _():
        wait_scatters(B)
    plsc.subcore_barrier()

    for q, accx in ((0, acc0), (1, acc1)):
        qg = c * 2 + q

        @pl.when(s < 15)
        def _():
            pltpu.sync_copy(accx.at[pl.ds(r0, TROWS)],
                            agg_out.at[qg, pl.ds(r0, TROWS)])

        @pl.when(s == 15)
        def _():
            pltpu.sync_copy(accx.at[pl.ds(LAST0, LASTN)],
                            agg_out.at[qg, pl.ds(LAST0, LASTN)])


_scat_pass = pl.kernel(
    _scat_body,
    out_type=jax.ShapeDtypeStruct((4, N, 16), jnp.float32),
    mesh=_MESH,
    compiler_params=pltpu.CompilerParams(use_tc_tiling_on_sc=False),
    scratch_types=(
        [pltpu.VMEM_SHARED((NPAD, 16), jnp.float32)] * 2
        + ([pltpu.VMEM((128,), jnp.int32)] * 4
           + [pltpu.VMEM((CH,), jnp.float32),
              pltpu.VMEM((2 * CH, 16), jnp.float32)]) * 2
        + [pltpu.SemaphoreType.DMA] * 6
    ),
)


def _mm0_body(x_ref, w_ref, b_ref, o_ref):
    o_ref[...] = jnp.maximum(x_ref[...], 0.0) @ w_ref[...] + b_ref[...]


def _tcb_body(hx_ref, w1_ref, w2_ref, dga_ref, dgb_ref, lwa_ref, lwb_ref,
              s0_out, s1_out, s2_out, s3_out, base_out, disv_out):
    hx = hx_ref[...]
    d0 = dga_ref[...]
    d1 = dgb_ref[...]
    l0 = lwa_ref[...]
    l1 = lwb_ref[...]
    lw = jnp.where(l1 >= 0, l1, jnp.where(l0 >= 0, l0, 1.0))
    deg = d0 + d1 + lw
    dis = jnp.where(deg > 0, lax.rsqrt(deg), 0.0)
    d2lw = dis * dis * lw
    u = C1 * hx + C2 * (hx @ w1_ref[...])
    us = dis * u
    s0_out[...] = us[:, 0:16]
    s1_out[...] = us[:, 16:32]
    s2_out[...] = us[:, 32:48]
    s3_out[...] = us[:, 48:64]
    base_out[...] = C3 * hx + C4 * (hx @ w2_ref[...]) + d2lw * u
    lanes = lax.broadcasted_iota(jnp.int32, (BM, 16), 1)
    disv_out[...] = jnp.where(lanes == 0, dis, jnp.where(lanes == 1, d2lw, 0.0))


def _tcd_body(hx_ref, base1_ref, a0_ref, a1_ref, a2_ref, a3_ref, disv_ref,
              w1_ref, w2_ref, s0_out, s1_out, s2_out, s3_out, base_out):
    dis = disv_ref[:, 0:1]
    d2lw = disv_ref[:, 1:2]
    agg = jnp.concatenate(
        [a0_ref[...], a1_ref[...], a2_ref[...], a3_ref[...]], axis=1)
    h1 = base1_ref[...] + dis * agg
    u = C1 * h1 + C2 * (h1 @ w1_ref[...])
    us = dis * u
    s0_out[...] = us[:, 0:16]
    s1_out[...] = us[:, 16:32]
    s2_out[...] = us[:, 32:48]
    s3_out[...] = us[:, 48:64]
    base_out[...] = C3 * hx_ref[...] + C4 * (hx_ref[...] @ w2_ref[...]) + d2lw * u


def _tce_body(base2_ref, a0_ref, a1_ref, a2_ref, a3_ref, disv_ref,
              w5_ref, b5_ref, o_ref):
    dis = disv_ref[:, 0:1]
    agg = jnp.concatenate(
        [a0_ref[...], a1_ref[...], a2_ref[...], a3_ref[...]], axis=1)
    h2 = jnp.maximum(base2_ref[...] + dis * agg, 0.0)
    o = h2 @ w5_ref[...] + b5_ref[...]
    m = jnp.max(o, axis=1, keepdims=True)
    z = o - m
    o_ref[...] = z - jnp.log(jnp.sum(jnp.exp(z), axis=1, keepdims=True))


def _rows_spec(width):
    return pl.BlockSpec((BM, width), lambda i: (i, 0))


def _full_spec(r, c_):
    return pl.BlockSpec((r, c_), lambda i: (0, 0))


def kernel(x, edge_index, edge_attr, W0, b0, W1a, W1b, W2a, W2b, W5, b5):
    row = edge_index[0]
    col = edge_index[1]

    hx = pl.pallas_call(
        _mm0_body,
        grid=(N // BM,),
        in_specs=[_rows_spec(FIN), _full_spec(FIN, H), _full_spec(1, H)],
        out_specs=_rows_spec(H),
        out_shape=jax.ShapeDtypeStruct((N, H), jnp.float32),
    )(x, W0, b0[None, :])

    z1 = jnp.zeros((TROWS,), jnp.float32)
    n1 = jnp.full((TROWS,), -1.0, jnp.float32)

    deg_a, deg_b, lw_a, lw_b = _deg_pass(row, col, edge_attr, z1, n1)
    dg0 = deg_a[:N].reshape(N, 1)
    dg1 = deg_b[:N].reshape(N, 1)
    lw0 = lw_a[:N].reshape(N, 1)
    lw1 = lw_b[:N].reshape(N, 1)

    q16 = jax.ShapeDtypeStruct((N, 16), jnp.float32)

    s10, s11, s12, s13, base1, disv = pl.pallas_call(
        _tcb_body,
        grid=(N // BM,),
        in_specs=[_rows_spec(H), _full_spec(H, H), _full_spec(H, H),
                  _rows_spec(1), _rows_spec(1), _rows_spec(1), _rows_spec(1)],
        out_specs=[_rows_spec(16)] * 4 + [_rows_spec(H), _rows_spec(16)],
        out_shape=[q16] * 4 + [jax.ShapeDtypeStruct((N, H), jnp.float32), q16],
    )(hx, W1a, W1b, dg0, dg1, lw0, lw1)

    agg1q = _scat_pass(row, col, edge_attr, s10, s11, s12, s13)

    s20, s21, s22, s23, base2 = pl.pallas_call(
        _tcd_body,
        grid=(N // BM,),
        in_specs=[_rows_spec(H), _rows_spec(H)] + [_rows_spec(16)] * 5
                 + [_full_spec(H, H), _full_spec(H, H)],
        out_specs=[_rows_spec(16)] * 4 + [_rows_spec(H)],
        out_shape=[q16] * 4 + [jax.ShapeDtypeStruct((N, H), jnp.float32)],
    )(hx, base1, agg1q[0], agg1q[1], agg1q[2], agg1q[3], disv, W2a, W2b)

    agg2q = _scat_pass(row, col, edge_attr, s20, s21, s22, s23)

    out = pl.pallas_call(
        _tce_body,
        grid=(N // BM,),
        in_specs=[_rows_spec(H)] + [_rows_spec(16)] * 5
                 + [_full_spec(H, C), _full_spec(1, C)],
        out_specs=_rows_spec(C),
        out_shape=jax.ShapeDtypeStruct((N, C), jnp.float32),
    )(base2, agg2q[0], agg2q[1], agg2q[2], agg2q[3], disv, W5, b5[None, :])

    return out


# speedup vs baseline: 1.0142x; 1.0142x over previous
import jax
import jax.numpy as jnp
from jax import lax
from jax.experimental import pallas as pl
from jax.experimental.pallas import tpu as pltpu
from jax.experimental.pallas import tpu_sc as plsc

N = 50000
E = 800000
FIN = 784
H = 64
C = 20
ALPHA = 0.2
BETA = 0.05
C1 = (1.0 - BETA) * (1.0 - ALPHA)
C2 = (1.0 - ALPHA) * BETA
C3 = (1.0 - BETA) * ALPHA
C4 = BETA * ALPHA

NPAD = 51200
TROWS = NPAD // 16
LAST0 = 15 * TROWS
LASTN = N - LAST0
DUMP_D = N
CH = 256
NCHUNK = E // CH
BM = 1000

_MESH = plsc.VectorSubcoreMesh(core_axis_name="c", subcore_axis_name="s")


def _splat(v, j):
    idx = jnp.full((16, 1), j, dtype=jnp.int32)
    dn = lax.GatherDimensionNumbers(
        offset_dims=(), collapsed_slice_dims=(0,), start_index_map=(0,))
    return lax.gather(v, idx, dn, (1,),
                      mode=lax.GatherScatterMode.PROMISE_IN_BOUNDS)


CHD = 640
NCHUNKD = E // CHD


def _deg_body(row_hbm, col_hbm, attr_hbm, z1_hbm, n1_hbm,
              dega_out, degb_out, lwa_out, lwb_out,
              deg_t, lw_t, rb0, rb1, rb2, rb3, rb4, colb, attrb, vdeg,
              il0, il1, il2, il3, il4, sem):
    c = lax.axis_index("c")
    s = lax.axis_index("s")
    w = s * 2 + c
    r0 = s * TROWS
    rbs = (rb0, rb1, rb2, rb3, rb4)
    ils = (il0, il1, il2, il3, il4)
    pltpu.sync_copy(z1_hbm, deg_t.at[pl.ds(r0, TROWS)])
    pltpu.sync_copy(n1_hbm, lw_t.at[pl.ds(r0, TROWS)])
    plsc.subcore_barrier()

    dumpv = jnp.full((16,), DUMP_D, jnp.int32)
    nt = (NCHUNKD - w + 31) // 32

    def chunk(k, _):
        off = (w + k * 32) * CHD
        for i in range(5):
            pltpu.async_copy(row_hbm.at[pl.ds(off + i * 128, 128)],
                             rbs[i], sem)
        pltpu.async_copy(col_hbm.at[pl.ds(off, CHD)], colb, sem)
        pltpu.async_copy(attr_hbm.at[pl.ds(off, CHD)], attrb, sem)
        for i in range(5):
            pltpu.make_async_copy(row_hbm.at[pl.ds(0, 128)],
                                  rbs[i], sem).wait()
        pltpu.make_async_copy(col_hbm.at[pl.ds(0, CHD)], colb, sem).wait()
        pltpu.make_async_copy(attr_hbm.at[pl.ds(0, CHD)], attrb, sem).wait()

        for g in range(CHD // 16):
            h = (g % 8) * 16
            rv = rbs[g // 8][pl.ds(h, 16)]
            cv = colb[pl.ds(g * 16, 16)]
            wv = attrb[pl.ds(g * 16, 16)]
            selfm = rv == cv
            ew = jnp.where(selfm, 0.0, wv)
            lwi = jnp.where(selfm, rv, dumpv)
            vdeg[pl.ds(g * 16, 16)] = ew
            ils[g // 8][pl.ds(h, 16)] = lwi

        dsc = []
        for i in range(5):
            dsc.append(pltpu.async_copy(vdeg.at[pl.ds(i * 128, 128)],
                                        deg_t.at[rbs[i]], sem, add=True))
            dsc.append(pltpu.async_copy(attrb.at[pl.ds(i * 128, 128)],
                                        lw_t.at[ils[i]], sem))
        for d in dsc:
            d.wait()
        return 0
    lax.fori_loop(0, nt, chunk, 0)
    plsc.subcore_barrier()

    @pl.when(c == 0)
    def _():
        pltpu.sync_copy(deg_t.at[pl.ds(r0, TROWS)],
                        dega_out.at[pl.ds(r0, TROWS)])
        pltpu.sync_copy(lw_t.at[pl.ds(r0, TROWS)],
                        lwa_out.at[pl.ds(r0, TROWS)])

    @pl.when(c == 1)
    def _():
        pltpu.sync_copy(deg_t.at[pl.ds(r0, TROWS)],
                        degb_out.at[pl.ds(r0, TROWS)])
        pltpu.sync_copy(lw_t.at[pl.ds(r0, TROWS)],
                        lwb_out.at[pl.ds(r0, TROWS)])


_deg_pass = pl.kernel(
    _deg_body,
    out_type=[jax.ShapeDtypeStruct((NPAD,), jnp.float32),
              jax.ShapeDtypeStruct((NPAD,), jnp.float32),
              jax.ShapeDtypeStruct((NPAD,), jnp.float32),
              jax.ShapeDtypeStruct((NPAD,), jnp.float32)],
    mesh=_MESH,
    scratch_types=(
        [pltpu.VMEM_SHARED((NPAD,), jnp.float32)] * 2
        + [pltpu.VMEM((128,), jnp.int32)] * 5
        + [pltpu.VMEM((CHD,), jnp.int32),
           pltpu.VMEM((CHD,), jnp.float32),
           pltpu.VMEM((CHD,), jnp.float32)]
        + [pltpu.VMEM((128,), jnp.int32)] * 5
        + [pltpu.SemaphoreType.DMA]
    ),
)


def _scat_body(row_hbm, col_hbm, attr_hbm, sup4_hbm, agg_out,
               acc0, acc1,
               rA0, rA1, cA0, cA1, gA00, gA01, gA10, gA11, aA, gbA,
               rB0, rB1, cB0, cB1, gB00, gB01, gB10, gB11, aB, gbB,
               semEA, semEB, semGA, semGB, semSA, semSB):
    c = lax.axis_index("c")
    s = lax.axis_index("s")
    r0 = s * TROWS
    qb0 = (c * 2) * N
    qb1 = (c * 2 + 1) * N

    A = (rA0, rA1, cA0, cA1, gA00, gA01, gA10, gA11, aA, gbA,
         semEA, semGA, semSA)
    B = (rB0, rB1, cB0, cB1, gB00, gB01, gB10, gB11, aB, gbB,
         semEB, semGB, semSB)

    def fire_edges(S, k):
        off = (s + k * 16) * CH
        pltpu.async_copy(row_hbm.at[pl.ds(off, 128)], S[0], S[10])
        pltpu.async_copy(row_hbm.at[pl.ds(off + 128, 128)], S[1], S[10])
        pltpu.async_copy(col_hbm.at[pl.ds(off, 128)], S[2], S[10])
        pltpu.async_copy(col_hbm.at[pl.ds(off + 128, 128)], S[3], S[10])
        pltpu.async_copy(attr_hbm.at[pl.ds(off, CH)], S[8], S[10])

    def wait_edges(S):
        pltpu.make_async_copy(row_hbm.at[pl.ds(0, 128)], S[0], S[10]).wait()
        pltpu.make_async_copy(row_hbm.at[pl.ds(0, 128)], S[1], S[10]).wait()
        pltpu.make_async_copy(col_hbm.at[pl.ds(0, 128)], S[2], S[10]).wait()
        pltpu.make_async_copy(col_hbm.at[pl.ds(0, 128)], S[3], S[10]).wait()
        pltpu.make_async_copy(attr_hbm.at[pl.ds(0, CH)], S[8], S[10]).wait()

    def fire_gathers(S):
        for g in range(8):
            h = g * 16
            S[4][pl.ds(h, 16)] = S[0][pl.ds(h, 16)] + qb0
            S[5][pl.ds(h, 16)] = S[1][pl.ds(h, 16)] + qb0
            S[6][pl.ds(h, 16)] = S[0][pl.ds(h, 16)] + qb1
            S[7][pl.ds(h, 16)] = S[1][pl.ds(h, 16)] + qb1
        gb = S[9]
        pltpu.async_copy(sup4_hbm.at[S[4]], gb.at[pl.ds(0, 128)], S[11])
        pltpu.async_copy(sup4_hbm.at[S[5]], gb.at[pl.ds(128, 128)], S[11])
        pltpu.async_copy(sup4_hbm.at[S[6]], gb.at[pl.ds(CH, 128)], S[11])
        pltpu.async_copy(sup4_hbm.at[S[7]], gb.at[pl.ds(CH + 128, 128)], S[11])

    def wait_gathers(S):
        gb = S[9]
        pltpu.make_async_copy(sup4_hbm.at[S[4]], gb.at[pl.ds(0, 128)], S[11]).wait()
        pltpu.make_async_copy(sup4_hbm.at[S[5]], gb.at[pl.ds(128, 128)], S[11]).wait()
        pltpu.make_async_copy(sup4_hbm.at[S[6]], gb.at[pl.ds(CH, 128)], S[11]).wait()
        pltpu.make_async_copy(sup4_hbm.at[S[7]], gb.at[pl.ds(CH + 128, 128)], S[11]).wait()

    def scale(S):
        gb = S[9]
        for g in range(16):
            h = (g % 8) * 16
            rv = (S[0] if g < 8 else S[1])[pl.ds(h, 16)]
            cv = (S[2] if g < 8 else S[3])[pl.ds(h, 16)]
            wv = S[8][pl.ds(g * 16, 16)]
            ew = jnp.where(rv == cv, 0.0, wv)
            for j in range(16):
                e = g * 16 + j
                sc_ = _splat(ew, j)
                gb[e] = gb[e, pl.ds(0, 16)] * sc_
                gb[CH + e] = gb[CH + e, pl.ds(0, 16)] * sc_

    def fire_scatters(S):
        gb = S[9]
        pltpu.async_copy(gb.at[pl.ds(0, 128)], acc0.at[S[2]], S[12], add=True)
        pltpu.async_copy(gb.at[pl.ds(128, 128)], acc0.at[S[3]], S[12], add=True)
        pltpu.async_copy(gb.at[pl.ds(CH, 128)], acc1.at[S[2]], S[12], add=True)
        pltpu.async_copy(gb.at[pl.ds(CH + 128, 128)], acc1.at[S[3]], S[12], add=True)

    def wait_scatters(S):
        gb = S[9]
        pltpu.make_async_copy(gb.at[pl.ds(0, 128)], acc0.at[S[2]], S[12]).wait()
        pltpu.make_async_copy(gb.at[pl.ds(128, 128)], acc0.at[S[3]], S[12]).wait()
        pltpu.make_async_copy(gb.at[pl.ds(CH, 128)], acc1.at[S[2]], S[12]).wait()
        pltpu.make_async_copy(gb.at[pl.ds(CH + 128, 128)], acc1.at[S[3]], S[12]).wait()

    zv = jnp.zeros((16,), jnp.float32)
    for r in range(256):
        gbA[r] = zv
    for accx in (acc0, acc1):
        for kk in range(12):
            pltpu.sync_copy(gbA.at[pl.ds(0, 256)],
                            accx.at[pl.ds(r0 + kk * 256, 256)])
        pltpu.sync_copy(gbA.at[pl.ds(0, 128)],
                        accx.at[pl.ds(r0 + 3072, 128)])
    plsc.subcore_barrier()

    nt = (NCHUNK - s + 15) // 16
    npair = nt // 2
    rem = nt - 2 * npair

    fire_edges(A, 0)

    def pairbody(p, _):
        k0 = 2 * p
        wait_edges(A)
        fire_gathers(A)

        @pl.when(k0 >= 1)
        def _():
            wait_scatters(B)
        fire_edges(B, k0 + 1)
        wait_gathers(A)
        scale(A)
        fire_scatters(A)

        wait_edges(B)
        fire_gathers(B)
        wait_scatters(A)

        @pl.when(k0 + 2 < nt)
        def _():
            fire_edges(A, k0 + 2)
        wait_gathers(B)
        scale(B)
        fire_scatters(B)
        return 0
    lax.fori_loop(0, npair, pairbody, 0)

    @pl.when(rem == 1)
    def _():
        wait_edges(A)
        fire_gathers(A)
        wait_scatters(B)
        wait_gathers(A)
        scale(A)
        fire_scatters(A)
        wait_scatters(A)

    @pl.when(rem == 0)
    def _():
        wait_scatters(B)
    plsc.subcore_barrier()

    for q, accx in ((0, acc0), (1, acc1)):
        qg = c * 2 + q

        @pl.when(s < 15)
        def _():
            pltpu.sync_copy(accx.at[pl.ds(r0, TROWS)],
                            agg_out.at[qg, pl.ds(r0, TROWS)])

        @pl.when(s == 15)
        def _():
            pltpu.sync_copy(accx.at[pl.ds(LAST0, LASTN)],
                            agg_out.at[qg, pl.ds(LAST0, LASTN)])


_scat_pass = pl.kernel(
    _scat_body,
    out_type=jax.ShapeDtypeStruct((4, N, 16), jnp.float32),
    mesh=_MESH,
    compiler_params=pltpu.CompilerParams(use_tc_tiling_on_sc=False),
    scratch_types=(
        [pltpu.VMEM_SHARED((NPAD, 16), jnp.float32)] * 2
        + ([pltpu.VMEM((128,), jnp.int32)] * 8
           + [pltpu.VMEM((CH,), jnp.float32),
              pltpu.VMEM((2 * CH, 16), jnp.float32)]) * 2
        + [pltpu.SemaphoreType.DMA] * 6
    ),
)


def _mm0_body(x_ref, w_ref, b_ref, o_ref):
    o_ref[...] = jnp.maximum(x_ref[...], 0.0) @ w_ref[...] + b_ref[...]


def _tcb_body(hx_ref, w1_ref, w2_ref, dga_ref, dgb_ref, lwa_ref, lwb_ref,
              sup_out, base_out, disv_out):
    hx = hx_ref[...]
    d0 = dga_ref[...]
    d1 = dgb_ref[...]
    l0 = lwa_ref[...]
    l1 = lwb_ref[...]
    lw = jnp.where(l1 >= 0, l1, jnp.where(l0 >= 0, l0, 1.0))
    deg = d0 + d1 + lw
    dis = jnp.where(deg > 0, lax.rsqrt(deg), 0.0)
    d2lw = dis * dis * lw
    u = C1 * hx + C2 * (hx @ w1_ref[...])
    sup_out[...] = dis * u
    base_out[...] = C3 * hx + C4 * (hx @ w2_ref[...]) + d2lw * u
    lanes = lax.broadcasted_iota(jnp.int32, (BM, 16), 1)
    disv_out[...] = jnp.where(lanes == 0, dis, jnp.where(lanes == 1, d2lw, 0.0))


def _tcd_body(hx_ref, base1_ref, agg_ref, disv_ref, w1_ref, w2_ref,
              sup_out, base_out):
    dis = disv_ref[:, 0:1]
    d2lw = disv_ref[:, 1:2]
    h1 = base1_ref[...] + dis * agg_ref[...]
    u = C1 * h1 + C2 * (h1 @ w1_ref[...])
    sup_out[...] = dis * u
    base_out[...] = C3 * hx_ref[...] + C4 * (hx_ref[...] @ w2_ref[...]) + d2lw * u


def _tce_body(base2_ref, agg_ref, disv_ref, w5_ref, b5_ref, o_ref):
    dis = disv_ref[:, 0:1]
    h2 = jnp.maximum(base2_ref[...] + dis * agg_ref[...], 0.0)
    o = h2 @ w5_ref[...] + b5_ref[...]
    m = jnp.max(o, axis=1, keepdims=True)
    z = o - m
    o_ref[...] = z - jnp.log(jnp.sum(jnp.exp(z), axis=1, keepdims=True))


def _rows_spec(width):
    return pl.BlockSpec((BM, width), lambda i: (i, 0))


def _full_spec(r, c_):
    return pl.BlockSpec((r, c_), lambda i: (0, 0))


def kernel(x, edge_index, edge_attr, W0, b0, W1a, W1b, W2a, W2b, W5, b5):
    row = edge_index[0]
    col = edge_index[1]

    hx = pl.pallas_call(
        _mm0_body,
        grid=(N // BM,),
        in_specs=[_rows_spec(FIN), _full_spec(FIN, H), _full_spec(1, H)],
        out_specs=_rows_spec(H),
        out_shape=jax.ShapeDtypeStruct((N, H), jnp.float32),
    )(x, W0, b0[None, :])

    z1 = jnp.zeros((TROWS,), jnp.float32)
    n1 = jnp.full((TROWS,), -1.0, jnp.float32)

    deg_a, deg_b, lw_a, lw_b = _deg_pass(row, col, edge_attr, z1, n1)
    dg0 = deg_a[:N].reshape(N, 1)
    dg1 = deg_b[:N].reshape(N, 1)
    lw0 = lw_a[:N].reshape(N, 1)
    lw1 = lw_b[:N].reshape(N, 1)

    sup1, base1, disv = pl.pallas_call(
        _tcb_body,
        grid=(N // BM,),
        in_specs=[_rows_spec(H), _full_spec(H, H), _full_spec(H, H),
                  _rows_spec(1), _rows_spec(1), _rows_spec(1), _rows_spec(1)],
        out_specs=[_rows_spec(H), _rows_spec(H), _rows_spec(16)],
        out_shape=[jax.ShapeDtypeStruct((N, H), jnp.float32),
                   jax.ShapeDtypeStruct((N, H), jnp.float32),
                   jax.ShapeDtypeStruct((N, 16), jnp.float32)],
    )(hx, W1a, W1b, dg0, dg1, lw0, lw1)

    sup1q = sup1.reshape(N, 4, 16).transpose(1, 0, 2).reshape(4 * N, 16)
    agg1q = _scat_pass(row, col, edge_attr, sup1q)
    agg1 = agg1q.transpose(1, 0, 2).reshape(N, H)

    sup2, base2 = pl.pallas_call(
        _tcd_body,
        grid=(N // BM,),
        in_specs=[_rows_spec(H), _rows_spec(H), _rows_spec(H), _rows_spec(16),
                  _full_spec(H, H), _full_spec(H, H)],
        out_specs=[_rows_spec(H), _rows_spec(H)],
        out_shape=[jax.ShapeDtypeStruct((N, H), jnp.float32),
                   jax.ShapeDtypeStruct((N, H), jnp.float32)],
    )(hx, base1, agg1, disv, W2a, W2b)

    sup2q = sup2.reshape(N, 4, 16).transpose(1, 0, 2).reshape(4 * N, 16)
    agg2q = _scat_pass(row, col, edge_attr, sup2q)
    agg2 = agg2q.transpose(1, 0, 2).reshape(N, H)

    out = pl.pallas_call(
        _tce_body,
        grid=(N // BM,),
        in_specs=[_rows_spec(H), _rows_spec(H), _rows_spec(16),
                  _full_spec(H, C), _full_spec(1, C)],
        out_specs=_rows_spec(C),
        out_shape=jax.ShapeDtypeStruct((N, C), jnp.float32),
    )(base2, agg2, disv, W5, b5[None, :])

    return out


# speedup vs baseline: 1.0693x; 1.0543x over previous
import jax
import jax.numpy as jnp
from jax import lax
from jax.experimental import pallas as pl
from jax.experimental.pallas import tpu as pltpu
from jax.experimental.pallas import tpu_sc as plsc

N = 50000
E = 800000
FIN = 784
H = 64
C = 20
ALPHA = 0.2
BETA = 0.05
C1 = (1.0 - BETA) * (1.0 - ALPHA)
C2 = (1.0 - ALPHA) * BETA
C3 = (1.0 - BETA) * ALPHA
C4 = BETA * ALPHA

NPAD = 51200
TROWS = NPAD // 16
LAST0 = 15 * TROWS
LASTN = N - LAST0
DUMP_D = N
CH = 256
NCHUNK = E // CH
BM = 1000

_MESH = plsc.VectorSubcoreMesh(core_axis_name="c", subcore_axis_name="s")


def _splat(v, j):
    idx = jnp.full((16, 1), j, dtype=jnp.int32)
    dn = lax.GatherDimensionNumbers(
        offset_dims=(), collapsed_slice_dims=(0,), start_index_map=(0,))
    return lax.gather(v, idx, dn, (1,),
                      mode=lax.GatherScatterMode.PROMISE_IN_BOUNDS)


CHD = 640
NCHUNKD = E // CHD


def _deg_body(row_hbm, col_hbm, attr_hbm, z1_hbm, n1_hbm,
              dega_out, degb_out, lwa_out, lwb_out,
              deg_t, lw_t, rb0, rb1, rb2, rb3, rb4, colb, attrb, vdeg,
              il0, il1, il2, il3, il4, sem):
    c = lax.axis_index("c")
    s = lax.axis_index("s")
    w = s * 2 + c
    r0 = s * TROWS
    rbs = (rb0, rb1, rb2, rb3, rb4)
    ils = (il0, il1, il2, il3, il4)
    pltpu.sync_copy(z1_hbm, deg_t.at[pl.ds(r0, TROWS)])
    pltpu.sync_copy(n1_hbm, lw_t.at[pl.ds(r0, TROWS)])
    plsc.subcore_barrier()

    dumpv = jnp.full((16,), DUMP_D, jnp.int32)
    nt = (NCHUNKD - w + 31) // 32

    def chunk(k, _):
        off = (w + k * 32) * CHD
        for i in range(5):
            pltpu.async_copy(row_hbm.at[pl.ds(off + i * 128, 128)],
                             rbs[i], sem)
        pltpu.async_copy(col_hbm.at[pl.ds(off, CHD)], colb, sem)
        pltpu.async_copy(attr_hbm.at[pl.ds(off, CHD)], attrb, sem)
        for i in range(5):
            pltpu.make_async_copy(row_hbm.at[pl.ds(0, 128)],
                                  rbs[i], sem).wait()
        pltpu.make_async_copy(col_hbm.at[pl.ds(0, CHD)], colb, sem).wait()
        pltpu.make_async_copy(attr_hbm.at[pl.ds(0, CHD)], attrb, sem).wait()

        for g in range(CHD // 16):
            h = (g % 8) * 16
            rv = rbs[g // 8][pl.ds(h, 16)]
            cv = colb[pl.ds(g * 16, 16)]
            wv = attrb[pl.ds(g * 16, 16)]
            selfm = rv == cv
            ew = jnp.where(selfm, 0.0, wv)
            lwi = jnp.where(selfm, rv, dumpv)
            vdeg[pl.ds(g * 16, 16)] = ew
            ils[g // 8][pl.ds(h, 16)] = lwi

        dsc = []
        for i in range(5):
            dsc.append(pltpu.async_copy(vdeg.at[pl.ds(i * 128, 128)],
                                        deg_t.at[rbs[i]], sem, add=True))
            dsc.append(pltpu.async_copy(attrb.at[pl.ds(i * 128, 128)],
                                        lw_t.at[ils[i]], sem))
        for d in dsc:
            d.wait()
        return 0
    lax.fori_loop(0, nt, chunk, 0)
    plsc.subcore_barrier()

    @pl.when(c == 0)
    def _():
        pltpu.sync_copy(deg_t.at[pl.ds(r0, TROWS)],
                        dega_out.at[pl.ds(r0, TROWS)])
        pltpu.sync_copy(lw_t.at[pl.ds(r0, TROWS)],
                        lwa_out.at[pl.ds(r0, TROWS)])

    @pl.when(c == 1)
    def _():
        pltpu.sync_copy(deg_t.at[pl.ds(r0, TROWS)],
                        degb_out.at[pl.ds(r0, TROWS)])
        pltpu.sync_copy(lw_t.at[pl.ds(r0, TROWS)],
                        lwb_out.at[pl.ds(r0, TROWS)])


_deg_pass = pl.kernel(
    _deg_body,
    out_type=[jax.ShapeDtypeStruct((NPAD,), jnp.float32),
              jax.ShapeDtypeStruct((NPAD,), jnp.float32),
              jax.ShapeDtypeStruct((NPAD,), jnp.float32),
              jax.ShapeDtypeStruct((NPAD,), jnp.float32)],
    mesh=_MESH,
    scratch_types=(
        [pltpu.VMEM_SHARED((NPAD,), jnp.float32)] * 2
        + [pltpu.VMEM((128,), jnp.int32)] * 5
        + [pltpu.VMEM((CHD,), jnp.int32),
           pltpu.VMEM((CHD,), jnp.float32),
           pltpu.VMEM((CHD,), jnp.float32)]
        + [pltpu.VMEM((128,), jnp.int32)] * 5
        + [pltpu.SemaphoreType.DMA]
    ),
)


def _scat_body(row_hbm, col_hbm, attr_hbm, sup4_hbm, agg_out,
               acc0, acc1,
               rA0, rA1, cA0, cA1, gA00, gA01, gA10, gA11, aA, gbA,
               rB0, rB1, cB0, cB1, gB00, gB01, gB10, gB11, aB, gbB,
               semEA, semEB, semGA, semGB, semSA, semSB):
    c = lax.axis_index("c")
    s = lax.axis_index("s")
    r0 = s * TROWS
    qb0 = (c * 2) * N
    qb1 = (c * 2 + 1) * N

    A = (rA0, rA1, cA0, cA1, gA00, gA01, gA10, gA11, aA, gbA,
         semEA, semGA, semSA)
    B = (rB0, rB1, cB0, cB1, gB00, gB01, gB10, gB11, aB, gbB,
         semEB, semGB, semSB)

    def fire_edges(S, k):
        off = (s + k * 16) * CH
        pltpu.async_copy(row_hbm.at[pl.ds(off, 128)], S[0], S[10])
        pltpu.async_copy(row_hbm.at[pl.ds(off + 128, 128)], S[1], S[10])
        pltpu.async_copy(col_hbm.at[pl.ds(off, 128)], S[2], S[10])
        pltpu.async_copy(col_hbm.at[pl.ds(off + 128, 128)], S[3], S[10])
        pltpu.async_copy(attr_hbm.at[pl.ds(off, CH)], S[8], S[10])

    def wait_edges(S):
        pltpu.make_async_copy(row_hbm.at[pl.ds(0, 128)], S[0], S[10]).wait()
        pltpu.make_async_copy(row_hbm.at[pl.ds(0, 128)], S[1], S[10]).wait()
        pltpu.make_async_copy(col_hbm.at[pl.ds(0, 128)], S[2], S[10]).wait()
        pltpu.make_async_copy(col_hbm.at[pl.ds(0, 128)], S[3], S[10]).wait()
        pltpu.make_async_copy(attr_hbm.at[pl.ds(0, CH)], S[8], S[10]).wait()

    def fire_gathers(S):
        for g in range(8):
            h = g * 16
            S[4][pl.ds(h, 16)] = S[0][pl.ds(h, 16)] + qb0
            S[5][pl.ds(h, 16)] = S[1][pl.ds(h, 16)] + qb0
            S[6][pl.ds(h, 16)] = S[0][pl.ds(h, 16)] + qb1
            S[7][pl.ds(h, 16)] = S[1][pl.ds(h, 16)] + qb1
        gb = S[9]
        pltpu.async_copy(sup4_hbm.at[S[4]], gb.at[pl.ds(0, 128)], S[11])
        pltpu.async_copy(sup4_hbm.at[S[5]], gb.at[pl.ds(128, 128)], S[11])
        pltpu.async_copy(sup4_hbm.at[S[6]], gb.at[pl.ds(CH, 128)], S[11])
        pltpu.async_copy(sup4_hbm.at[S[7]], gb.at[pl.ds(CH + 128, 128)], S[11])

    def wait_gathers(S):
        gb = S[9]
        pltpu.make_async_copy(sup4_hbm.at[S[4]], gb.at[pl.ds(0, 128)], S[11]).wait()
        pltpu.make_async_copy(sup4_hbm.at[S[5]], gb.at[pl.ds(128, 128)], S[11]).wait()
        pltpu.make_async_copy(sup4_hbm.at[S[6]], gb.at[pl.ds(CH, 128)], S[11]).wait()
        pltpu.make_async_copy(sup4_hbm.at[S[7]], gb.at[pl.ds(CH + 128, 128)], S[11]).wait()

    def scale(S):
        gb = S[9]
        for g in range(16):
            h = (g % 8) * 16
            rv = (S[0] if g < 8 else S[1])[pl.ds(h, 16)]
            cv = (S[2] if g < 8 else S[3])[pl.ds(h, 16)]
            wv = S[8][pl.ds(g * 16, 16)]
            ew = jnp.where(rv == cv, 0.0, wv)
            for j in range(16):
                e = g * 16 + j
                sc_ = _splat(ew, j)
                gb[e] = gb[e, pl.ds(0, 16)] * sc_
                gb[CH + e] = gb[CH + e, pl.ds(0, 16)] * sc_

    def fire_scatters(S):
        gb = S[9]
        pltpu.async_copy(gb.at[pl.ds(0, 128)], acc0.at[S[2]], S[12], add=True)
        pltpu.async_copy(gb.at[pl.ds(128, 128)], acc0.at[S[3]], S[12], add=True)
        pltpu.async_copy(gb.at[pl.ds(CH, 128)], acc1.at[S[2]], S[12], add=True)
        pltpu.async_copy(gb.at[pl.ds(CH + 128, 128)], acc1.at[S[3]], S[12], add=True)

    def wait_scatters(S):
        gb = S[9]
        pltpu.make_async_copy(gb.at[pl.ds(0, 128)], acc0.at[S[2]], S[12]).wait()
        pltpu.make_async_copy(gb.at[pl.ds(128, 128)], acc0.at[S[3]], S[12]).wait()
        pltpu.make_async_copy(gb.at[pl.ds(CH, 128)], acc1.at[S[2]], S[12]).wait()
        pltpu.make_async_copy(gb.at[pl.ds(CH + 128, 128)], acc1.at[S[3]], S[12]).wait()

    zv = jnp.zeros((16,), jnp.float32)
    for r in range(256):
        gbA[r] = zv
    for accx in (acc0, acc1):
        for kk in range(12):
            pltpu.sync_copy(gbA.at[pl.ds(0, 256)],
                            accx.at[pl.ds(r0 + kk * 256, 256)])
        pltpu.sync_copy(gbA.at[pl.ds(0, 128)],
                        accx.at[pl.ds(r0 + 3072, 128)])
    plsc.subcore_barrier()

    nt = (NCHUNK - s + 15) // 16
    npair = nt // 2
    rem = nt - 2 * npair

    fire_edges(A, 0)

    def pairbody(p, _):
        k0 = 2 * p
        wait_edges(A)
        fire_gathers(A)

        @pl.when(k0 >= 1)
        def _():
            wait_scatters(B)
        fire_edges(B, k0 + 1)
        wait_gathers(A)
        scale(A)
        fire_scatters(A)

        wait_edges(B)
        fire_gathers(B)
        wait_scatters(A)

        @pl.when(k0 + 2 < nt)
        def _():
            fire_edges(A, k0 + 2)
        wait_gathers(B)
        scale(B)
        fire_scatters(B)
        return 0
    lax.fori_loop(0, npair, pairbody, 0)

    @pl.when(rem == 1)
    def _():
        wait_edges(A)
        fire_gathers(A)
        wait_scatters(B)
        wait_gathers(A)
        scale(A)
        fire_scatters(A)
        wait_scatters(A)

    @pl.when(rem == 0)
    def _():
        wait_scatters(B)
    plsc.subcore_barrier()

    for q, accx in ((0, acc0), (1, acc1)):
        qg = c * 2 + q

        @pl.when(s < 15)
        def _():
            pltpu.sync_copy(accx.at[pl.ds(r0, TROWS)],
                            agg_out.at[qg, pl.ds(r0, TROWS)])

        @pl.when(s == 15)
        def _():
            pltpu.sync_copy(accx.at[pl.ds(LAST0, LASTN)],
                            agg_out.at[qg, pl.ds(LAST0, LASTN)])


_scat_pass = pl.kernel(
    _scat_body,
    out_type=jax.ShapeDtypeStruct((4, N, 16), jnp.float32),
    mesh=_MESH,
    compiler_params=pltpu.CompilerParams(use_tc_tiling_on_sc=False),
    scratch_types=(
        [pltpu.VMEM_SHARED((NPAD, 16), jnp.float32)] * 2
        + ([pltpu.VMEM((128,), jnp.int32)] * 8
           + [pltpu.VMEM((CH,), jnp.float32),
              pltpu.VMEM((2 * CH, 16), jnp.float32)]) * 2
        + [pltpu.SemaphoreType.DMA] * 6
    ),
)


def _mm0_body(x_ref, w_ref, b_ref, o_ref):
    o_ref[...] = jnp.maximum(x_ref[...], 0.0) @ w_ref[...] + b_ref[...]


def _tcb_body(hx_ref, w1_ref, w2_ref, disv2_ref,
              sup_out, base_out, disv_out):
    hx = hx_ref[...]
    dis = disv2_ref[:, 0:1]
    d2lw = disv2_ref[:, 1:2]
    u = C1 * hx + C2 * (hx @ w1_ref[...])
    sup_out[...] = dis * u
    base_out[...] = C3 * hx + C4 * (hx @ w2_ref[...]) + d2lw * u
    lanes = lax.broadcasted_iota(jnp.int32, (BM, 16), 1)
    disv_out[...] = jnp.where(lanes == 0, dis, jnp.where(lanes == 1, d2lw, 0.0))


def _tcd_body(hx_ref, base1_ref, agg_ref, disv_ref, w1_ref, w2_ref,
              sup_out, base_out):
    dis = disv_ref[:, 0:1]
    d2lw = disv_ref[:, 1:2]
    h1 = base1_ref[...] + dis * agg_ref[...]
    u = C1 * h1 + C2 * (h1 @ w1_ref[...])
    sup_out[...] = dis * u
    base_out[...] = C3 * hx_ref[...] + C4 * (hx_ref[...] @ w2_ref[...]) + d2lw * u


def _tce_body(base2_ref, agg_ref, disv_ref, w5_ref, b5_ref, o_ref):
    dis = disv_ref[:, 0:1]
    h2 = jnp.maximum(base2_ref[...] + dis * agg_ref[...], 0.0)
    o = h2 @ w5_ref[...] + b5_ref[...]
    m = jnp.max(o, axis=1, keepdims=True)
    z = o - m
    o_ref[...] = z - jnp.log(jnp.sum(jnp.exp(z), axis=1, keepdims=True))


def _rows_spec(width):
    return pl.BlockSpec((BM, width), lambda i: (i, 0))


def _full_spec(r, c_):
    return pl.BlockSpec((r, c_), lambda i: (0, 0))


def kernel(x, edge_index, edge_attr, W0, b0, W1a, W1b, W2a, W2b, W5, b5):
    row = edge_index[0]
    col = edge_index[1]

    hx = pl.pallas_call(
        _mm0_body,
        grid=(N // BM,),
        in_specs=[_rows_spec(FIN), _full_spec(FIN, H), _full_spec(1, H)],
        out_specs=_rows_spec(H),
        out_shape=jax.ShapeDtypeStruct((N, H), jnp.float32),
    )(x, W0, b0[None, :])

    z1 = jnp.zeros((TROWS,), jnp.float32)
    n1 = jnp.full((TROWS,), -1.0, jnp.float32)

    deg_a, deg_b, lw_a, lw_b = _deg_pass(row, col, edge_attr, z1, n1)
    lw = jnp.where(lw_b >= 0, lw_b, jnp.where(lw_a >= 0, lw_a, 1.0))
    deg = deg_a + deg_b + lw
    dis = jnp.where(deg > 0, deg ** -0.5, 0.0)
    d2lw = dis * dis * lw
    disv2 = jnp.concatenate(
        [dis[:N].reshape(N, 1), d2lw[:N].reshape(N, 1)], axis=1)

    sup1, base1, disv = pl.pallas_call(
        _tcb_body,
        grid=(N // BM,),
        in_specs=[_rows_spec(H), _full_spec(H, H), _full_spec(H, H),
                  _rows_spec(2)],
        out_specs=[_rows_spec(H), _rows_spec(H), _rows_spec(16)],
        out_shape=[jax.ShapeDtypeStruct((N, H), jnp.float32),
                   jax.ShapeDtypeStruct((N, H), jnp.float32),
                   jax.ShapeDtypeStruct((N, 16), jnp.float32)],
    )(hx, W1a, W1b, disv2)

    sup1q = sup1.reshape(N, 4, 16).transpose(1, 0, 2).reshape(4 * N, 16)
    agg1q = _scat_pass(row, col, edge_attr, sup1q)
    agg1 = agg1q.transpose(1, 0, 2).reshape(N, H)

    sup2, base2 = pl.pallas_call(
        _tcd_body,
        grid=(N // BM,),
        in_specs=[_rows_spec(H), _rows_spec(H), _rows_spec(H), _rows_spec(16),
                  _full_spec(H, H), _full_spec(H, H)],
        out_specs=[_rows_spec(H), _rows_spec(H)],
        out_shape=[jax.ShapeDtypeStruct((N, H), jnp.float32),
                   jax.ShapeDtypeStruct((N, H), jnp.float32)],
    )(hx, base1, agg1, disv, W2a, W2b)

    sup2q = sup2.reshape(N, 4, 16).transpose(1, 0, 2).reshape(4 * N, 16)
    agg2q = _scat_pass(row, col, edge_attr, sup2q)
    agg2 = agg2q.transpose(1, 0, 2).reshape(N, H)

    out = pl.pallas_call(
        _tce_body,
        grid=(N // BM,),
        in_specs=[_rows_spec(H), _rows_spec(H), _rows_spec(16),
                  _full_spec(H, C), _full_spec(1, C)],
        out_specs=_rows_spec(C),
        out_shape=jax.ShapeDtypeStruct((N, C), jnp.float32),
    )(base2, agg2, disv, W5, b5[None, :])

    return out


# speedup vs baseline: 1.1994x; 1.1217x over previous
import jax
import jax.numpy as jnp
from jax import lax
from jax.experimental import pallas as pl
from jax.experimental.pallas import tpu as pltpu
from jax.experimental.pallas import tpu_sc as plsc

N = 50000
E = 800000
FIN = 784
H = 64
C = 20
ALPHA = 0.2
BETA = 0.05
C1 = (1.0 - BETA) * (1.0 - ALPHA)
C2 = (1.0 - ALPHA) * BETA
C3 = (1.0 - BETA) * ALPHA
C4 = BETA * ALPHA

NPAD = 51200
TROWS = NPAD // 16
LAST0 = 15 * TROWS
LASTN = N - LAST0
DUMP_D = N
CH = 256
NCHUNK = E // CH
BM = 1000

_MESH = plsc.VectorSubcoreMesh(core_axis_name="c", subcore_axis_name="s")


def _splat(v, j):
    idx = jnp.full((16, 1), j, dtype=jnp.int32)
    dn = lax.GatherDimensionNumbers(
        offset_dims=(), collapsed_slice_dims=(0,), start_index_map=(0,))
    return lax.gather(v, idx, dn, (1,),
                      mode=lax.GatherScatterMode.PROMISE_IN_BOUNDS)


CHD = 640
NCHUNKD = E // CHD


def _deg_body(row_hbm, col_hbm, attr_hbm, z1_hbm, n1_hbm,
              dega_out, degb_out, lwa_out, lwb_out,
              deg_t, lw_t, rb0, rb1, rb2, rb3, rb4, colb, attrb, vdeg,
              il0, il1, il2, il3, il4, sem):
    c = lax.axis_index("c")
    s = lax.axis_index("s")
    w = s * 2 + c
    r0 = s * TROWS
    rbs = (rb0, rb1, rb2, rb3, rb4)
    ils = (il0, il1, il2, il3, il4)
    pltpu.sync_copy(z1_hbm, deg_t.at[pl.ds(r0, TROWS)])
    pltpu.sync_copy(n1_hbm, lw_t.at[pl.ds(r0, TROWS)])
    plsc.subcore_barrier()

    dumpv = jnp.full((16,), DUMP_D, jnp.int32)
    nt = (NCHUNKD - w + 31) // 32

    def chunk(k, _):
        off = (w + k * 32) * CHD
        for i in range(5):
            pltpu.async_copy(row_hbm.at[pl.ds(off + i * 128, 128)],
                             rbs[i], sem)
        pltpu.async_copy(col_hbm.at[pl.ds(off, CHD)], colb, sem)
        pltpu.async_copy(attr_hbm.at[pl.ds(off, CHD)], attrb, sem)
        for i in range(5):
            pltpu.make_async_copy(row_hbm.at[pl.ds(0, 128)],
                                  rbs[i], sem).wait()
        pltpu.make_async_copy(col_hbm.at[pl.ds(0, CHD)], colb, sem).wait()
        pltpu.make_async_copy(attr_hbm.at[pl.ds(0, CHD)], attrb, sem).wait()

        for g in range(CHD // 16):
            h = (g % 8) * 16
            rv = rbs[g // 8][pl.ds(h, 16)]
            cv = colb[pl.ds(g * 16, 16)]
            wv = attrb[pl.ds(g * 16, 16)]
            selfm = rv == cv
            ew = jnp.where(selfm, 0.0, wv)
            lwi = jnp.where(selfm, rv, dumpv)
            vdeg[pl.ds(g * 16, 16)] = ew
            ils[g // 8][pl.ds(h, 16)] = lwi

        dsc = []
        for i in range(5):
            dsc.append(pltpu.async_copy(vdeg.at[pl.ds(i * 128, 128)],
                                        deg_t.at[rbs[i]], sem, add=True))
            dsc.append(pltpu.async_copy(attrb.at[pl.ds(i * 128, 128)],
                                        lw_t.at[ils[i]], sem))
        for d in dsc:
            d.wait()
        return 0
    lax.fori_loop(0, nt, chunk, 0)
    plsc.subcore_barrier()

    @pl.when(c == 0)
    def _():
        pltpu.sync_copy(deg_t.at[pl.ds(r0, TROWS)],
                        dega_out.at[pl.ds(r0, TROWS)])
        pltpu.sync_copy(lw_t.at[pl.ds(r0, TROWS)],
                        lwa_out.at[pl.ds(r0, TROWS)])

    @pl.when(c == 1)
    def _():
        pltpu.sync_copy(deg_t.at[pl.ds(r0, TROWS)],
                        degb_out.at[pl.ds(r0, TROWS)])
        pltpu.sync_copy(lw_t.at[pl.ds(r0, TROWS)],
                        lwb_out.at[pl.ds(r0, TROWS)])


_deg_pass = pl.kernel(
    _deg_body,
    out_type=[jax.ShapeDtypeStruct((NPAD,), jnp.float32),
              jax.ShapeDtypeStruct((NPAD,), jnp.float32),
              jax.ShapeDtypeStruct((NPAD,), jnp.float32),
              jax.ShapeDtypeStruct((NPAD,), jnp.float32)],
    mesh=_MESH,
    scratch_types=(
        [pltpu.VMEM_SHARED((NPAD,), jnp.float32)] * 2
        + [pltpu.VMEM((128,), jnp.int32)] * 5
        + [pltpu.VMEM((CHD,), jnp.int32),
           pltpu.VMEM((CHD,), jnp.float32),
           pltpu.VMEM((CHD,), jnp.float32)]
        + [pltpu.VMEM((128,), jnp.int32)] * 5
        + [pltpu.SemaphoreType.DMA]
    ),
)


def _scat_body(row_hbm, col_hbm, attr_hbm, sup4_hbm, agg_out,
               acc0, acc1,
               rA0, rA1, cA0, cA1, gA00, gA01, gA10, gA11, aA, gbA,
               rB0, rB1, cB0, cB1, gB00, gB01, gB10, gB11, aB, gbB,
               semEA, semEB, semGA, semGB, semSA, semSB):
    c = lax.axis_index("c")
    s = lax.axis_index("s")
    r0 = s * TROWS
    qb0 = c * 2
    qb1 = c * 2 + 1

    A = (rA0, rA1, cA0, cA1, gA00, gA01, gA10, gA11, aA, gbA,
         semEA, semGA, semSA)
    B = (rB0, rB1, cB0, cB1, gB00, gB01, gB10, gB11, aB, gbB,
         semEB, semGB, semSB)

    def fire_edges(S, k):
        off = (s + k * 16) * CH
        pltpu.async_copy(row_hbm.at[pl.ds(off, 128)], S[0], S[10])
        pltpu.async_copy(row_hbm.at[pl.ds(off + 128, 128)], S[1], S[10])
        pltpu.async_copy(col_hbm.at[pl.ds(off, 128)], S[2], S[10])
        pltpu.async_copy(col_hbm.at[pl.ds(off + 128, 128)], S[3], S[10])
        pltpu.async_copy(attr_hbm.at[pl.ds(off, CH)], S[8], S[10])

    def wait_edges(S):
        pltpu.make_async_copy(row_hbm.at[pl.ds(0, 128)], S[0], S[10]).wait()
        pltpu.make_async_copy(row_hbm.at[pl.ds(0, 128)], S[1], S[10]).wait()
        pltpu.make_async_copy(col_hbm.at[pl.ds(0, 128)], S[2], S[10]).wait()
        pltpu.make_async_copy(col_hbm.at[pl.ds(0, 128)], S[3], S[10]).wait()
        pltpu.make_async_copy(attr_hbm.at[pl.ds(0, CH)], S[8], S[10]).wait()

    def fire_gathers(S):
        for g in range(8):
            h = g * 16
            r4a = S[0][pl.ds(h, 16)] * 4
            r4b = S[1][pl.ds(h, 16)] * 4
            S[4][pl.ds(h, 16)] = r4a + qb0
            S[5][pl.ds(h, 16)] = r4b + qb0
            S[6][pl.ds(h, 16)] = r4a + qb1
            S[7][pl.ds(h, 16)] = r4b + qb1
        gb = S[9]
        pltpu.async_copy(sup4_hbm.at[S[4]], gb.at[pl.ds(0, 128)], S[11])
        pltpu.async_copy(sup4_hbm.at[S[5]], gb.at[pl.ds(128, 128)], S[11])
        pltpu.async_copy(sup4_hbm.at[S[6]], gb.at[pl.ds(CH, 128)], S[11])
        pltpu.async_copy(sup4_hbm.at[S[7]], gb.at[pl.ds(CH + 128, 128)], S[11])

    def wait_gathers(S):
        gb = S[9]
        pltpu.make_async_copy(sup4_hbm.at[S[4]], gb.at[pl.ds(0, 128)], S[11]).wait()
        pltpu.make_async_copy(sup4_hbm.at[S[5]], gb.at[pl.ds(128, 128)], S[11]).wait()
        pltpu.make_async_copy(sup4_hbm.at[S[6]], gb.at[pl.ds(CH, 128)], S[11]).wait()
        pltpu.make_async_copy(sup4_hbm.at[S[7]], gb.at[pl.ds(CH + 128, 128)], S[11]).wait()

    def scale(S):
        gb = S[9]
        for g in range(16):
            h = (g % 8) * 16
            rv = (S[0] if g < 8 else S[1])[pl.ds(h, 16)]
            cv = (S[2] if g < 8 else S[3])[pl.ds(h, 16)]
            wv = S[8][pl.ds(g * 16, 16)]
            ew = jnp.where(rv == cv, 0.0, wv)
            for j in range(16):
                e = g * 16 + j
                sc_ = _splat(ew, j)
                gb[e] = gb[e, pl.ds(0, 16)] * sc_
                gb[CH + e] = gb[CH + e, pl.ds(0, 16)] * sc_

    def fire_scatters(S):
        gb = S[9]
        pltpu.async_copy(gb.at[pl.ds(0, 128)], acc0.at[S[2]], S[12], add=True)
        pltpu.async_copy(gb.at[pl.ds(128, 128)], acc0.at[S[3]], S[12], add=True)
        pltpu.async_copy(gb.at[pl.ds(CH, 128)], acc1.at[S[2]], S[12], add=True)
        pltpu.async_copy(gb.at[pl.ds(CH + 128, 128)], acc1.at[S[3]], S[12], add=True)

    def wait_scatters(S):
        gb = S[9]
        pltpu.make_async_copy(gb.at[pl.ds(0, 128)], acc0.at[S[2]], S[12]).wait()
        pltpu.make_async_copy(gb.at[pl.ds(128, 128)], acc0.at[S[3]], S[12]).wait()
        pltpu.make_async_copy(gb.at[pl.ds(CH, 128)], acc1.at[S[2]], S[12]).wait()
        pltpu.make_async_copy(gb.at[pl.ds(CH + 128, 128)], acc1.at[S[3]], S[12]).wait()

    zv = jnp.zeros((16,), jnp.float32)
    for r in range(256):
        gbA[r] = zv
    for accx in (acc0, acc1):
        for kk in range(12):
            pltpu.sync_copy(gbA.at[pl.ds(0, 256)],
                            accx.at[pl.ds(r0 + kk * 256, 256)])
        pltpu.sync_copy(gbA.at[pl.ds(0, 128)],
                        accx.at[pl.ds(r0 + 3072, 128)])
    plsc.subcore_barrier()

    nt = (NCHUNK - s + 15) // 16
    npair = nt // 2
    rem = nt - 2 * npair

    fire_edges(A, 0)

    def pairbody(p, _):
        k0 = 2 * p
        wait_edges(A)
        fire_gathers(A)

        @pl.when(k0 >= 1)
        def _():
            wait_scatters(B)
        fire_edges(B, k0 + 1)
        wait_gathers(A)
        scale(A)
        fire_scatters(A)

        wait_edges(B)
        fire_gathers(B)
        wait_scatters(A)

        @pl.when(k0 + 2 < nt)
        def _():
            fire_edges(A, k0 + 2)
        wait_gathers(B)
        scale(B)
        fire_scatters(B)
        return 0
    lax.fori_loop(0, npair, pairbody, 0)

    @pl.when(rem == 1)
    def _():
        wait_edges(A)
        fire_gathers(A)
        wait_scatters(B)
        wait_gathers(A)
        scale(A)
        fire_scatters(A)
        wait_scatters(A)

    @pl.when(rem == 0)
    def _():
        wait_scatters(B)
    plsc.subcore_barrier()

    for q, accx in ((0, acc0), (1, acc1)):
        qg = c * 2 + q

        @pl.when(s < 15)
        def _():
            pltpu.sync_copy(accx.at[pl.ds(r0, TROWS)],
                            agg_out.at[qg, pl.ds(r0, TROWS)])

        @pl.when(s == 15)
        def _():
            pltpu.sync_copy(accx.at[pl.ds(LAST0, LASTN)],
                            agg_out.at[qg, pl.ds(LAST0, LASTN)])


_scat_pass = pl.kernel(
    _scat_body,
    out_type=jax.ShapeDtypeStruct((4, N, 16), jnp.float32),
    mesh=_MESH,
    compiler_params=pltpu.CompilerParams(use_tc_tiling_on_sc=False),
    scratch_types=(
        [pltpu.VMEM_SHARED((NPAD, 16), jnp.float32)] * 2
        + ([pltpu.VMEM((128,), jnp.int32)] * 8
           + [pltpu.VMEM((CH,), jnp.float32),
              pltpu.VMEM((2 * CH, 16), jnp.float32)]) * 2
        + [pltpu.SemaphoreType.DMA] * 6
    ),
)


def _mm0_body(x_ref, w_ref, b_ref, o_ref):
    o_ref[...] = jnp.maximum(x_ref[...], 0.0) @ w_ref[...] + b_ref[...]


def _tcb_body(hx_ref, w1_ref, w2_ref, disv2_ref,
              sup_out, base_out, disv_out):
    hx = hx_ref[...]
    dis = disv2_ref[:, 0:1]
    d2lw = disv2_ref[:, 1:2]
    u = C1 * hx + C2 * (hx @ w1_ref[...])
    sup_out[...] = dis * u
    base_out[...] = C3 * hx + C4 * (hx @ w2_ref[...]) + d2lw * u
    lanes = lax.broadcasted_iota(jnp.int32, (BM, 16), 1)
    disv_out[...] = jnp.where(lanes == 0, dis, jnp.where(lanes == 1, d2lw, 0.0))


def _tcd_body(hx_ref, base1_ref, agg_ref, disv_ref, w1_ref, w2_ref,
              sup_out, base_out):
    dis = disv_ref[:, 0:1]
    d2lw = disv_ref[:, 1:2]
    h1 = base1_ref[...] + dis * agg_ref[...]
    u = C1 * h1 + C2 * (h1 @ w1_ref[...])
    sup_out[...] = dis * u
    base_out[...] = C3 * hx_ref[...] + C4 * (hx_ref[...] @ w2_ref[...]) + d2lw * u


def _tce_body(base2_ref, agg_ref, disv_ref, w5_ref, b5_ref, o_ref):
    dis = disv_ref[:, 0:1]
    h2 = jnp.maximum(base2_ref[...] + dis * agg_ref[...], 0.0)
    o = h2 @ w5_ref[...] + b5_ref[...]
    m = jnp.max(o, axis=1, keepdims=True)
    z = o - m
    o_ref[...] = z - jnp.log(jnp.sum(jnp.exp(z), axis=1, keepdims=True))


def _rows_spec(width):
    return pl.BlockSpec((BM, width), lambda i: (i, 0))


def _full_spec(r, c_):
    return pl.BlockSpec((r, c_), lambda i: (0, 0))


def kernel(x, edge_index, edge_attr, W0, b0, W1a, W1b, W2a, W2b, W5, b5):
    row = edge_index[0]
    col = edge_index[1]

    hx = pl.pallas_call(
        _mm0_body,
        grid=(N // BM,),
        in_specs=[_rows_spec(FIN), _full_spec(FIN, H), _full_spec(1, H)],
        out_specs=_rows_spec(H),
        out_shape=jax.ShapeDtypeStruct((N, H), jnp.float32),
    )(x, W0, b0[None, :])

    z1 = jnp.zeros((TROWS,), jnp.float32)
    n1 = jnp.full((TROWS,), -1.0, jnp.float32)

    deg_a, deg_b, lw_a, lw_b = _deg_pass(row, col, edge_attr, z1, n1)
    lw = jnp.where(lw_b >= 0, lw_b, jnp.where(lw_a >= 0, lw_a, 1.0))
    deg = deg_a + deg_b + lw
    dis = jnp.where(deg > 0, deg ** -0.5, 0.0)
    d2lw = dis * dis * lw
    disv2 = jnp.concatenate(
        [dis[:N].reshape(N, 1), d2lw[:N].reshape(N, 1)], axis=1)

    sup1, base1, disv = pl.pallas_call(
        _tcb_body,
        grid=(N // BM,),
        in_specs=[_rows_spec(H), _full_spec(H, H), _full_spec(H, H),
                  _rows_spec(2)],
        out_specs=[_rows_spec(H), _rows_spec(H), _rows_spec(16)],
        out_shape=[jax.ShapeDtypeStruct((N, H), jnp.float32),
                   jax.ShapeDtypeStruct((N, H), jnp.float32),
                   jax.ShapeDtypeStruct((N, 16), jnp.float32)],
    )(hx, W1a, W1b, disv2)

    sup1q = sup1.reshape(4 * N, 16)
    agg1q = _scat_pass(row, col, edge_attr, sup1q)
    agg1 = agg1q.transpose(1, 0, 2).reshape(N, H)

    sup2, base2 = pl.pallas_call(
        _tcd_body,
        grid=(N // BM,),
        in_specs=[_rows_spec(H), _rows_spec(H), _rows_spec(H), _rows_spec(16),
                  _full_spec(H, H), _full_spec(H, H)],
        out_specs=[_rows_spec(H), _rows_spec(H)],
        out_shape=[jax.ShapeDtypeStruct((N, H), jnp.float32),
                   jax.ShapeDtypeStruct((N, H), jnp.float32)],
    )(hx, base1, agg1, disv, W2a, W2b)

    sup2q = sup2.reshape(4 * N, 16)
    agg2q = _scat_pass(row, col, edge_attr, sup2q)
    agg2 = agg2q.transpose(1, 0, 2).reshape(N, H)

    out = pl.pallas_call(
        _tce_body,
        grid=(N // BM,),
        in_specs=[_rows_spec(H), _rows_spec(H), _rows_spec(16),
                  _full_spec(H, C), _full_spec(1, C)],
        out_specs=_rows_spec(C),
        out_shape=jax.ShapeDtypeStruct((N, C), jnp.float32),
    )(base2, agg2, disv, W5, b5[None, :])

    return out
